# baseline (speedup 1.0000x reference)
import functools

import jax
import jax.numpy as jnp
from jax import lax
from jax.experimental import pallas as pl
from jax.experimental.pallas import tpu as pltpu
from jax.experimental.pallas import tpu_sc as plsc

BLK = 1024
HID = 64
KTOP = 8
NCORES = 2
NSUBCORES = 16
NWORKERS = NCORES * NSUBCORES
LANES = 16
ROW_UNROLL = 8
SC_CHUNK = 128


def _fused_tc_kernel(data_ref, w_ref, scale_ref, out_ref):
    act = jnp.dot(data_ref[...], w_ref[...],
                  preferred_element_type=jnp.float32)
    a = act.T
    for _ in range(KTOP - 1):
        m = jnp.max(a, axis=0, keepdims=True)
        a = jnp.where(a == m, -jnp.inf, a)
    thr = jnp.max(a, axis=0, keepdims=True)
    thr_col = thr.reshape(BLK, 1)
    scale = scale_ref[0]
    out_ref[...] = jnp.where(act >= thr_col, scale, jnp.float32(0.0))


def _fused_tc(data, W, scale):
    B = data.shape[0]
    return pl.pallas_call(
        _fused_tc_kernel,
        grid=(B // BLK,),
        in_specs=[
            pl.BlockSpec((BLK, data.shape[1]), lambda i: (i, 0)),
            pl.BlockSpec((W.shape[0], W.shape[1]), lambda i: (0, 0)),
            pl.BlockSpec(memory_space=pltpu.SMEM),
        ],
        out_specs=pl.BlockSpec((BLK, HID), lambda i: (i, 0)),
        out_shape=jax.ShapeDtypeStruct((B, HID), jnp.float32),
        compiler_params=pltpu.CompilerParams(
            dimension_semantics=("arbitrary",),
        ),
    )(data, W, scale)


def _mm_kernel(data_ref, w_ref, act_ref):
    act_ref[...] = jnp.dot(data_ref[...], w_ref[...],
                           preferred_element_type=jnp.float32)


def _tc_matmul(data, W):
    B = data.shape[0]
    return pl.pallas_call(
        _mm_kernel,
        grid=(B // BLK,),
        in_specs=[
            pl.BlockSpec((BLK, data.shape[1]), lambda i: (i, 0)),
            pl.BlockSpec((W.shape[0], W.shape[1]), lambda i: (0, 0)),
        ],
        out_specs=pl.BlockSpec((BLK, HID), lambda i: (i, 0)),
        out_shape=jax.ShapeDtypeStruct((B, HID), jnp.float32),
        compiler_params=pltpu.CompilerParams(
            dimension_semantics=("arbitrary",),
        ),
    )(data, W)


def _top8_threshold(a0, a1, a2, a3):
    neg = jnp.float32(-jnp.inf)
    thr = jnp.float32(0.0)
    for r in range(KTOP):
        m = jnp.maximum(jnp.maximum(a0, a1), jnp.maximum(a2, a3))
        thr = jnp.max(m)
        if r < KTOP - 1:
            a0 = jnp.where(a0 == thr, neg, a0)
            a1 = jnp.where(a1 == thr, neg, a1)
            a2 = jnp.where(a2 == thr, neg, a2)
            a3 = jnp.where(a3 == thr, neg, a3)
    return thr


def _sc_body(act_hbm, scale_hbm, h_hbm,
             a0_v, a1_v, h0_v, h1_v, scale_v,
             sem_in0, sem_in1, sem_out0, sem_out1):
    rows_per_w = act_hbm.shape[0] // NWORKERS
    wid = lax.axis_index("s") * NCORES + lax.axis_index("c")
    base = wid * rows_per_w
    pltpu.sync_copy(scale_hbm, scale_v)
    sv = scale_v[...]
    zero = jnp.float32(0.0)
    bufs = [(a0_v, h0_v, sem_in0, sem_out0), (a1_v, h1_v, sem_in1, sem_out1)]
    nch = rows_per_w // SC_CHUNK

    def one_row(act_v, h_v, r):
        a0 = act_v[r, pl.ds(0, LANES)]
        a1 = act_v[r, pl.ds(16, LANES)]
        a2 = act_v[r, pl.ds(32, LANES)]
        a3 = act_v[r, pl.ds(48, LANES)]
        thr = _top8_threshold(a0, a1, a2, a3)
        h_v[r, pl.ds(0, LANES)] = jnp.where(a0 >= thr, sv, zero)
        h_v[r, pl.ds(16, LANES)] = jnp.where(a1 >= thr, sv, zero)
        h_v[r, pl.ds(32, LANES)] = jnp.where(a2 >= thr, sv, zero)
        h_v[r, pl.ds(48, LANES)] = jnp.where(a3 >= thr, sv, zero)

    in_copies = [None] * nch
    out_copies = [None, None]
    in_copies[0] = pltpu.async_copy(
        act_hbm.at[pl.ds(base, SC_CHUNK)], a0_v, sem_in0)
    for c in range(nch):
        act_v, h_v, _, so = bufs[c % 2]
        in_copies[c].wait()
        if c + 1 < nch:
            nav, _, sin, _ = bufs[(c + 1) % 2]
            in_copies[c + 1] = pltpu.async_copy(
                act_hbm.at[pl.ds(base + (c + 1) * SC_CHUNK, SC_CHUNK)],
                nav, sin)
        if out_copies[c % 2] is not None:
            out_copies[c % 2].wait()

        def rows_body(i, carry):
            for u in range(ROW_UNROLL):
                one_row(act_v, h_v, i * ROW_UNROLL + u)
            return carry

        lax.fori_loop(0, SC_CHUNK // ROW_UNROLL, rows_body, 0)
        out_copies[c % 2] = pltpu.async_copy(
            h_v, h_hbm.at[pl.ds(base + c * SC_CHUNK, SC_CHUNK)], so)
    out_copies[(nch - 1) % 2].wait()
    if nch > 1:
        out_copies[nch % 2].wait()


def _sc_top8(act, scale):
    B = act.shape[0]
    rows_per_w = B // NWORKERS
    scale_vec = jnp.full((LANES,), scale, dtype=jnp.float32)
    mesh = plsc.VectorSubcoreMesh(
        core_axis_name="c", subcore_axis_name="s")
    fn = functools.partial(
        pl.kernel,
        out_type=jax.ShapeDtypeStruct((B, HID), jnp.float32),
        mesh=mesh,
        scratch_types=[
            pltpu.VMEM((SC_CHUNK, HID), jnp.float32),
            pltpu.VMEM((SC_CHUNK, HID), jnp.float32),
            pltpu.VMEM((SC_CHUNK, HID), jnp.float32),
            pltpu.VMEM((SC_CHUNK, HID), jnp.float32),
            pltpu.VMEM((LANES,), jnp.float32),
            pltpu.SemaphoreType.DMA,
            pltpu.SemaphoreType.DMA,
            pltpu.SemaphoreType.DMA,
            pltpu.SemaphoreType.DMA,
        ],
        compiler_params=pltpu.CompilerParams(needs_layout_passes=False),
    )(_sc_body)
    return fn(act, scale_vec)


@jax.jit
def kernel(data, W, k):
    scale = (jnp.asarray(k) - (KTOP - 1)).astype(jnp.float32).reshape(1)
    act = _tc_matmul(data, W)
    return _sc_top8(act, scale[0])


# speedup vs baseline: 1.1226x; 1.1226x over previous
import functools

import jax
import jax.numpy as jnp
from jax import lax
from jax.experimental import pallas as pl
from jax.experimental.pallas import tpu as pltpu
from jax.experimental.pallas import tpu_sc as plsc

BLK = 1024
HID = 64
KTOP = 8
NCORES = 2
NSUBCORES = 16
NWORKERS = NCORES * NSUBCORES
LANES = 16
ROW_UNROLL = 4
SC_CHUNK = 128


def _fused_tc_kernel(data_ref, w_ref, scale_ref, out_ref):
    act = jnp.dot(data_ref[...], w_ref[...],
                  preferred_element_type=jnp.float32)
    a = act.T
    for _ in range(KTOP - 1):
        m = jnp.max(a, axis=0, keepdims=True)
        a = jnp.where(a == m, -jnp.inf, a)
    thr = jnp.max(a, axis=0, keepdims=True)
    thr_col = thr.reshape(BLK, 1)
    scale = scale_ref[0]
    out_ref[...] = jnp.where(act >= thr_col, scale, jnp.float32(0.0))


def _fused_tc(data, W, scale):
    B = data.shape[0]
    return pl.pallas_call(
        _fused_tc_kernel,
        grid=(B // BLK,),
        in_specs=[
            pl.BlockSpec((BLK, data.shape[1]), lambda i: (i, 0)),
            pl.BlockSpec((W.shape[0], W.shape[1]), lambda i: (0, 0)),
            pl.BlockSpec(memory_space=pltpu.SMEM),
        ],
        out_specs=pl.BlockSpec((BLK, HID), lambda i: (i, 0)),
        out_shape=jax.ShapeDtypeStruct((B, HID), jnp.float32),
        compiler_params=pltpu.CompilerParams(
            dimension_semantics=("arbitrary",),
        ),
    )(data, W, scale)


def _mm_kernel(data_ref, w_ref, act_ref):
    act_ref[...] = jnp.dot(data_ref[...], w_ref[...],
                           preferred_element_type=jnp.float32)


def _tc_matmul(data, W):
    B = data.shape[0]
    return pl.pallas_call(
        _mm_kernel,
        grid=(B // BLK,),
        in_specs=[
            pl.BlockSpec((BLK, data.shape[1]), lambda i: (i, 0)),
            pl.BlockSpec((W.shape[0], W.shape[1]), lambda i: (0, 0)),
        ],
        out_specs=pl.BlockSpec((BLK, HID), lambda i: (i, 0)),
        out_shape=jax.ShapeDtypeStruct((B, HID), jnp.float32),
        compiler_params=pltpu.CompilerParams(
            dimension_semantics=("arbitrary",),
        ),
    )(data, W)


def _top8_threshold(a0, a1, a2, a3):
    neg = jnp.float32(-jnp.inf)
    thr = jnp.float32(0.0)
    for r in range(KTOP):
        m = jnp.maximum(jnp.maximum(a0, a1), jnp.maximum(a2, a3))
        thr = jnp.max(m)
        if r < KTOP - 1:
            a0 = jnp.where(a0 == thr, neg, a0)
            a1 = jnp.where(a1 == thr, neg, a1)
            a2 = jnp.where(a2 == thr, neg, a2)
            a3 = jnp.where(a3 == thr, neg, a3)
    return thr


def _sc_body(act_hbm, scale_hbm, h_hbm,
             a0_v, a1_v, h0_v, h1_v, scale_v,
             sem_in0, sem_in1, sem_out0, sem_out1):
    rows_per_w = act_hbm.shape[0] // NWORKERS
    wid = lax.axis_index("s") * NCORES + lax.axis_index("c")
    base = wid * rows_per_w
    pltpu.sync_copy(scale_hbm, scale_v)
    sv = scale_v[...]
    zero = jnp.float32(0.0)
    bufs = [(a0_v, h0_v, sem_in0, sem_out0), (a1_v, h1_v, sem_in1, sem_out1)]
    nch = rows_per_w // SC_CHUNK

    def one_row(act_v, h_v, r):
        a0 = act_v[r, pl.ds(0, LANES)]
        a1 = act_v[r, pl.ds(16, LANES)]
        a2 = act_v[r, pl.ds(32, LANES)]
        a3 = act_v[r, pl.ds(48, LANES)]
        thr = _top8_threshold(a0, a1, a2, a3)
        h_v[r, pl.ds(0, LANES)] = jnp.where(a0 >= thr, sv, zero)
        h_v[r, pl.ds(16, LANES)] = jnp.where(a1 >= thr, sv, zero)
        h_v[r, pl.ds(32, LANES)] = jnp.where(a2 >= thr, sv, zero)
        h_v[r, pl.ds(48, LANES)] = jnp.where(a3 >= thr, sv, zero)

    in_copies = [None] * nch
    out_copies = [None, None]
    in_copies[0] = pltpu.async_copy(
        act_hbm.at[pl.ds(base, SC_CHUNK)], a0_v, sem_in0)
    for c in range(nch):
        act_v, h_v, _, so = bufs[c % 2]
        in_copies[c].wait()
        if c + 1 < nch:
            nav, _, sin, _ = bufs[(c + 1) % 2]
            in_copies[c + 1] = pltpu.async_copy(
                act_hbm.at[pl.ds(base + (c + 1) * SC_CHUNK, SC_CHUNK)],
                nav, sin)
        if out_copies[c % 2] is not None:
            out_copies[c % 2].wait()

        def rows_body(i, carry):
            for u in range(ROW_UNROLL):
                one_row(act_v, h_v, i * ROW_UNROLL + u)
            return carry

        lax.fori_loop(0, SC_CHUNK // ROW_UNROLL, rows_body, 0)
        out_copies[c % 2] = pltpu.async_copy(
            h_v, h_hbm.at[pl.ds(base + c * SC_CHUNK, SC_CHUNK)], so)
    out_copies[(nch - 1) % 2].wait()
    if nch > 1:
        out_copies[nch % 2].wait()


def _sc_top8(act, scale):
    B = act.shape[0]
    rows_per_w = B // NWORKERS
    scale_vec = jnp.full((LANES,), scale, dtype=jnp.float32)
    mesh = plsc.VectorSubcoreMesh(
        core_axis_name="c", subcore_axis_name="s")
    fn = functools.partial(
        pl.kernel,
        out_type=jax.ShapeDtypeStruct((B, HID), jnp.float32),
        mesh=mesh,
        scratch_types=[
            pltpu.VMEM((SC_CHUNK, HID), jnp.float32),
            pltpu.VMEM((SC_CHUNK, HID), jnp.float32),
            pltpu.VMEM((SC_CHUNK, HID), jnp.float32),
            pltpu.VMEM((SC_CHUNK, HID), jnp.float32),
            pltpu.VMEM((LANES,), jnp.float32),
            pltpu.SemaphoreType.DMA,
            pltpu.SemaphoreType.DMA,
            pltpu.SemaphoreType.DMA,
            pltpu.SemaphoreType.DMA,
        ],
        compiler_params=pltpu.CompilerParams(needs_layout_passes=False),
    )(_sc_body)
    return fn(act, scale_vec)


@jax.jit
def kernel(data, W, k):
    scale = (jnp.asarray(k) - (KTOP - 1)).astype(jnp.float32).reshape(1)
    act = _tc_matmul(data, W)
    return _sc_top8(act, scale[0])


# speedup vs baseline: 1.3343x; 1.1885x over previous
import functools

import jax
import jax.numpy as jnp
from jax import lax
from jax.experimental import pallas as pl
from jax.experimental.pallas import tpu as pltpu
from jax.experimental.pallas import tpu_sc as plsc

BLK = 1024
HID = 64
KTOP = 8
NCORES = 2
NSUBCORES = 16
NWORKERS = NCORES * NSUBCORES
LANES = 16
ROW_UNROLL = 4
SC_CHUNK = 128


def _top8_rows_tc(act, scale):
    blk = act.shape[0]
    a = act.T
    for _ in range(KTOP - 1):
        m = jnp.max(a, axis=0, keepdims=True)
        a = jnp.where(a == m, -jnp.inf, a)
    thr = jnp.max(a, axis=0, keepdims=True)
    thr_col = thr.reshape(blk, 1)
    return jnp.where(act >= thr_col, scale, jnp.float32(0.0))


def _fused_tc_kernel(data0_ref, data1_ref, w_ref, scale_ref, out_ref):
    w = w_ref[...]
    scale = scale_ref[0]
    act0 = jnp.dot(data0_ref[...], w, preferred_element_type=jnp.float32)
    out_ref[0:BLK, :] = _top8_rows_tc(act0, scale)
    act1 = jnp.dot(data1_ref[...], w, preferred_element_type=jnp.float32)
    out_ref[BLK:2 * BLK, :] = _top8_rows_tc(act1, scale)


def _fused_tc(data, W, scale):
    B = data.shape[0]
    return pl.pallas_call(
        _fused_tc_kernel,
        grid=(B // (2 * BLK),),
        in_specs=[
            pl.BlockSpec((BLK, data.shape[1]), lambda i: (2 * i, 0)),
            pl.BlockSpec((BLK, data.shape[1]), lambda i: (2 * i + 1, 0)),
            pl.BlockSpec((W.shape[0], W.shape[1]), lambda i: (0, 0)),
            pl.BlockSpec(memory_space=pltpu.SMEM),
        ],
        out_specs=pl.BlockSpec((2 * BLK, HID), lambda i: (i, 0)),
        out_shape=jax.ShapeDtypeStruct((B, HID), jnp.float32),
        compiler_params=pltpu.CompilerParams(
            dimension_semantics=("arbitrary",),
        ),
    )(data, data, W, scale)


def _mm_kernel(data_ref, w_ref, act_ref):
    act_ref[...] = jnp.dot(data_ref[...], w_ref[...],
                           preferred_element_type=jnp.float32)


def _tc_matmul(data, W):
    B = data.shape[0]
    return pl.pallas_call(
        _mm_kernel,
        grid=(B // BLK,),
        in_specs=[
            pl.BlockSpec((BLK, data.shape[1]), lambda i: (i, 0)),
            pl.BlockSpec((W.shape[0], W.shape[1]), lambda i: (0, 0)),
        ],
        out_specs=pl.BlockSpec((BLK, HID), lambda i: (i, 0)),
        out_shape=jax.ShapeDtypeStruct((B, HID), jnp.float32),
        compiler_params=pltpu.CompilerParams(
            dimension_semantics=("arbitrary",),
        ),
    )(data, W)


def _top8_threshold(a0, a1, a2, a3):
    neg = jnp.float32(-jnp.inf)
    thr = jnp.float32(0.0)
    for r in range(KTOP):
        m = jnp.maximum(jnp.maximum(a0, a1), jnp.maximum(a2, a3))
        thr = jnp.max(m)
        if r < KTOP - 1:
            a0 = jnp.where(a0 == thr, neg, a0)
            a1 = jnp.where(a1 == thr, neg, a1)
            a2 = jnp.where(a2 == thr, neg, a2)
            a3 = jnp.where(a3 == thr, neg, a3)
    return thr


def _sc_body(act_hbm, scale_hbm, h_hbm,
             a0_v, a1_v, h0_v, h1_v, scale_v,
             sem_in0, sem_in1, sem_out0, sem_out1):
    rows_per_w = act_hbm.shape[0] // NWORKERS
    wid = lax.axis_index("s") * NCORES + lax.axis_index("c")
    base = wid * rows_per_w
    pltpu.sync_copy(scale_hbm, scale_v)
    sv = scale_v[...]
    zero = jnp.float32(0.0)
    bufs = [(a0_v, h0_v, sem_in0, sem_out0), (a1_v, h1_v, sem_in1, sem_out1)]
    nch = rows_per_w // SC_CHUNK

    def one_row(act_v, h_v, r):
        a0 = act_v[r, pl.ds(0, LANES)]
        a1 = act_v[r, pl.ds(16, LANES)]
        a2 = act_v[r, pl.ds(32, LANES)]
        a3 = act_v[r, pl.ds(48, LANES)]
        thr = _top8_threshold(a0, a1, a2, a3)
        h_v[r, pl.ds(0, LANES)] = jnp.where(a0 >= thr, sv, zero)
        h_v[r, pl.ds(16, LANES)] = jnp.where(a1 >= thr, sv, zero)
        h_v[r, pl.ds(32, LANES)] = jnp.where(a2 >= thr, sv, zero)
        h_v[r, pl.ds(48, LANES)] = jnp.where(a3 >= thr, sv, zero)

    in_copies = [None] * nch
    out_copies = [None, None]
    in_copies[0] = pltpu.async_copy(
        act_hbm.at[pl.ds(base, SC_CHUNK)], a0_v, sem_in0)
    for c in range(nch):
        act_v, h_v, _, so = bufs[c % 2]
        in_copies[c].wait()
        if c + 1 < nch:
            nav, _, sin, _ = bufs[(c + 1) % 2]
            in_copies[c + 1] = pltpu.async_copy(
                act_hbm.at[pl.ds(base + (c + 1) * SC_CHUNK, SC_CHUNK)],
                nav, sin)
        if out_copies[c % 2] is not None:
            out_copies[c % 2].wait()

        def rows_body(i, carry):
            for u in range(ROW_UNROLL):
                one_row(act_v, h_v, i * ROW_UNROLL + u)
            return carry

        lax.fori_loop(0, SC_CHUNK // ROW_UNROLL, rows_body, 0)
        out_copies[c % 2] = pltpu.async_copy(
            h_v, h_hbm.at[pl.ds(base + c * SC_CHUNK, SC_CHUNK)], so)
    out_copies[(nch - 1) % 2].wait()
    if nch > 1:
        out_copies[nch % 2].wait()


def _sc_top8(act, scale):
    B = act.shape[0]
    rows_per_w = B // NWORKERS
    scale_vec = jnp.full((LANES,), scale, dtype=jnp.float32)
    mesh = plsc.VectorSubcoreMesh(
        core_axis_name="c", subcore_axis_name="s")
    fn = functools.partial(
        pl.kernel,
        out_type=jax.ShapeDtypeStruct((B, HID), jnp.float32),
        mesh=mesh,
        scratch_types=[
            pltpu.VMEM((SC_CHUNK, HID), jnp.float32),
            pltpu.VMEM((SC_CHUNK, HID), jnp.float32),
            pltpu.VMEM((SC_CHUNK, HID), jnp.float32),
            pltpu.VMEM((SC_CHUNK, HID), jnp.float32),
            pltpu.VMEM((LANES,), jnp.float32),
            pltpu.SemaphoreType.DMA,
            pltpu.SemaphoreType.DMA,
            pltpu.SemaphoreType.DMA,
            pltpu.SemaphoreType.DMA,
        ],
        compiler_params=pltpu.CompilerParams(needs_layout_passes=False),
    )(_sc_body)
    return fn(act, scale_vec)


@jax.jit
def kernel(data, W, k):
    scale = (jnp.asarray(k) - (KTOP - 1)).astype(jnp.float32).reshape(1)
    return _fused_tc(data, W, scale)
